# baseline (speedup 1.0000x reference)
import functools

import jax
import jax.numpy as jnp
from jax import lax
from jax.experimental import pallas as pl
from jax.experimental.pallas import tpu as pltpu
from jax.experimental.pallas import tpu_sc as plsc

_LN_EPS = 1e-5
_B, _C, _N = 32, 96, 16384
_K = 4096
_L = 16
_SEG = _N // _L
_NB1 = 512
_HS = 512
_LEVELS = [(18, 32), (13, 32), (8, 32), (4, 16), (0, 16)]
_NCORES = 2


def _sc_body(pooled_hbm, fm_hbm, params_hbm, out_hbm,
             row0, row1, lista, listb, hist,
             salbuf, pooledbuf, parbuf, outbuf, sem0, sem1):
    iota = lax.iota(jnp.int32, _L)
    lane_hist = iota * _HS
    lane_seg = iota * _SEG
    zero16 = jnp.zeros((_L,), jnp.int32)
    zero16f = jnp.zeros((_L,), jnp.float32)
    ones16 = jnp.ones((_L,), jnp.int32)
    lane0 = iota == 0

    b = lax.axis_index("s") * _NCORES + lax.axis_index("c")

    def bcast_i(x):
        return lax.broadcast_in_dim(x, (_L,), ())

    def bcast_f(x):
        return lax.broadcast_in_dim(x, (_L,), ())

    def key_of(s):
        return s ^ ((s >> 31) & jnp.int32(0x7FFFFFFF))

    def select_bucket(nb, k_cur, total):
        carry = jnp.int32(0)
        na = jnp.int32(0)
        nge = jnp.int32(0)
        for ci in range(nb // _L):
            acc = zero16
            for l in range(_L):
                acc = acc + hist[pl.ds(l * _HS + ci * _L, _L)]
            cum = plsc.cumsum(acc) + carry
            suffix = total - cum + acc
            ge = suffix >= k_cur
            nge = nge + jnp.sum(ge.astype(jnp.int32))
            na = na + jnp.sum(jnp.where(ge, 0, acc))
            carry = carry + jnp.sum(acc)
        return nge - 1, na

    def zero_hist(nb):
        for l in range(_L):
            for ci in range(nb // _L):
                hist[pl.ds(l * _HS + ci * _L, _L)] = zero16

    def level_list(src, dst, shift, nb, lane_cnt, k_cur, sa):
        total = jnp.sum(lane_cnt)
        mx = jnp.max(lane_cnt)
        zero_hist(nb)
        mask_b = jnp.int32(nb - 1)

        def hbody(j, _):
            key = plsc.load_gather(src, [lane_seg + j])
            valid = j < lane_cnt
            bucket = (key >> shift) & mask_b
            plsc.addupdate_scatter(hist, [lane_hist + bucket], ones16,
                                   mask=valid)
            return 0

        lax.fori_loop(0, mx, hbody, 0)
        bstar, na = select_bucket(nb, k_cur, total)

        def cbody(j, st):
            cnt, sa = st
            key = plsc.load_gather(src, [lane_seg + j])
            valid = j < lane_cnt
            v = plsc.bitcast(key_of(key), jnp.float32)
            bucket = (key >> shift) & mask_b
            gt = jnp.logical_and(bucket > bstar, valid)
            sa = sa + jnp.where(gt, v, 0.0)
            eq = jnp.logical_and(bucket == bstar, valid)
            plsc.store_scatter(dst, [lane_seg + cnt], key, mask=eq)
            cnt = cnt + eq.astype(jnp.int32)
            return cnt, sa

        cnt, sa = lax.fori_loop(0, mx, cbody, (zero16, sa))
        return bstar, cnt, k_cur - na, sa

    def select_row(row_ref, c):
        zero_hist(_NB1)

        def h1(j, _):
            x = row_ref[pl.ds(j * _L, _L)]
            s = plsc.bitcast(x, jnp.int32)
            bucket = (key_of(s) >> 23) + 256
            plsc.addupdate_scatter(hist, [lane_hist + bucket], ones16)
            return 0

        lax.fori_loop(0, _N // _L, h1, 0)
        bstar1, na1 = select_bucket(_NB1, jnp.int32(_K), jnp.int32(_N))

        def c1(j, st):
            cnt, sa = st
            x = row_ref[pl.ds(j * _L, _L)]
            s = plsc.bitcast(x, jnp.int32)
            bucket = (key_of(s) >> 23) + 256
            gt = bucket > bstar1
            sa = sa + jnp.where(gt, x, 0.0)
            eq = bucket == bstar1
            plsc.store_scatter(lista, [lane_seg + cnt], key_of(s), mask=eq)
            cnt = cnt + eq.astype(jnp.int32)
            return cnt, sa

        cnt, sa = lax.fori_loop(0, _N // _L, c1, (zero16, zero16f))
        kc = jnp.int32(_K) - na1
        k_acc = (bstar1 - 256) << 23
        src, dst = lista, listb
        for shift, nb in _LEVELS:
            bst, cnt, kc, sa = level_list(src, dst, shift, nb, cnt, kc, sa)
            k_acc = k_acc + (bst << shift)
            src, dst = dst, src

        vk = plsc.bitcast(key_of(bcast_i(k_acc)), jnp.float32)
        ties = jnp.where(lane0, bcast_f(kc.astype(jnp.float32)) * vk, 0.0)
        sal = jnp.sum((sa + ties) * (1.0 / _K))
        plsc.store_scatter(salbuf, [bcast_i(c)], bcast_f(sal), mask=lane0)

    pltpu.make_async_copy(fm_hbm.at[b, 0], row0, sem0).start()
    pltpu.sync_copy(pooled_hbm.at[b], pooledbuf)
    pltpu.sync_copy(params_hbm, parbuf)

    def rowpair(i, _):
        c0 = i * 2
        pltpu.make_async_copy(fm_hbm.at[b, c0], row0, sem0).wait()
        cp1 = pltpu.make_async_copy(fm_hbm.at[b, c0 + 1], row1, sem1)
        cp1.start()
        select_row(row0, c0)
        cp1.wait()

        @pl.when(i < _C // 2 - 1)
        def _():
            pltpu.make_async_copy(fm_hbm.at[b, c0 + 2], row0, sem0).start()

        select_row(row1, c0 + 1)
        return 0

    lax.fori_loop(0, _C // 2, rowpair, 0)

    def rsqrt16(v):
        i = plsc.bitcast(v, jnp.int32)
        y = plsc.bitcast(jnp.int32(0x5F3759DF) - (i >> 1), jnp.float32)
        for _ in range(4):
            y = y * (1.5 - 0.5 * v * y * y)
        return y

    nch = _C // _L

    def stats(ref):
        accv = zero16f
        for ci in range(nch):
            accv = accv + ref[pl.ds(ci * _L, _L)]
        mean = bcast_f(jnp.sum(accv) * (1.0 / _C))
        varv = zero16f
        for ci in range(nch):
            d = ref[pl.ds(ci * _L, _L)] - mean
            varv = varv + d * d
        rstd = rsqrt16(bcast_f(jnp.sum(varv) * (1.0 / _C)) + _LN_EPS)
        return mean, rstd

    mean1, rstd1 = stats(salbuf)
    for ci in range(nch):
        sl = pl.ds(ci * _L, _L)
        w1 = parbuf[pl.ds(0 * _C + ci * _L, _L)]
        b1 = parbuf[pl.ds(1 * _C + ci * _L, _L)]
        gs = parbuf[pl.ds(2 * _C + ci * _L, _L)]
        gb = parbuf[pl.ds(3 * _C + ci * _L, _L)]
        rs = parbuf[pl.ds(4 * _C + ci * _L, _L)]
        sal = (salbuf[sl] - mean1) * rstd1 * w1 + b1
        pld = pooledbuf[sl]
        gate = 1.0 / (1.0 + jnp.exp(-(gs * sal + gb)))
        outbuf[sl] = pld + rs * gate * (sal - pld)

    mean2, rstd2 = stats(outbuf)
    for ci in range(nch):
        sl = pl.ds(ci * _L, _L)
        w2 = parbuf[pl.ds(5 * _C + ci * _L, _L)]
        b2 = parbuf[pl.ds(6 * _C + ci * _L, _L)]
        outbuf[sl] = (outbuf[sl] - mean2) * rstd2 * w2 + b2

    pltpu.sync_copy(outbuf, out_hbm.at[b])


_sc_kernel = functools.partial(
    pl.kernel,
    mesh=plsc.VectorSubcoreMesh(core_axis_name="c", subcore_axis_name="s"),
    out_type=jax.ShapeDtypeStruct((_B, _C), jnp.float32),
    compiler_params=pltpu.CompilerParams(needs_layout_passes=False),
    scratch_types=[
        pltpu.VMEM((_N,), jnp.float32),
        pltpu.VMEM((_N,), jnp.float32),
        pltpu.VMEM((_N,), jnp.int32),
        pltpu.VMEM((_N,), jnp.int32),
        pltpu.VMEM((_L * _HS,), jnp.int32),
        pltpu.VMEM((_C,), jnp.float32),
        pltpu.VMEM((_C,), jnp.float32),
        pltpu.VMEM((7 * _C,), jnp.float32),
        pltpu.VMEM((_C,), jnp.float32),
        pltpu.SemaphoreType.DMA,
        pltpu.SemaphoreType.DMA,
    ],
)(_sc_body)


def kernel(pooled, feature_map, ln1_w, ln1_b, gate_scale, gate_bias,
           residual_scale, ln2_w, ln2_b):
    fm = feature_map.astype(jnp.float32).reshape(_B, _C, _N)
    params = jnp.concatenate([
        ln1_w, ln1_b, gate_scale, gate_bias, residual_scale, ln2_w, ln2_b,
    ]).astype(jnp.float32)
    out = _sc_kernel(pooled.astype(jnp.float32), fm, params)
    return out.astype(pooled.dtype)


# speedup vs baseline: 2.1679x; 2.1679x over previous
import functools

import jax
import jax.numpy as jnp
from jax import lax
from jax.experimental import pallas as pl
from jax.experimental.pallas import tpu as pltpu
from jax.experimental.pallas import tpu_sc as plsc

_LN_EPS = 1e-5
_B, _C, _N = 32, 96, 16384
_K = 4096
_L = 16
_SEG = _N // _L
_NB1 = 512
_HS = 512
_LEVELS = [(18, 32), (13, 32), (8, 32), (4, 16), (0, 16)]
_NCORES = 2


def _sc_body(pooled_hbm, fm_hbm, params_hbm, out_hbm,
             row0, row1, lista, listb, hist,
             salbuf, pooledbuf, parbuf, outbuf, sem0, sem1):
    iota = lax.iota(jnp.int32, _L)
    lane_hist = iota * _HS
    lane_seg = iota * _SEG
    zero16 = jnp.zeros((_L,), jnp.int32)
    zero16f = jnp.zeros((_L,), jnp.float32)
    ones16 = jnp.ones((_L,), jnp.int32)
    lane0 = iota == 0

    b = lax.axis_index("s") * _NCORES + lax.axis_index("c")

    def bcast_i(x):
        return lax.broadcast_in_dim(x, (_L,), ())

    def bcast_f(x):
        return lax.broadcast_in_dim(x, (_L,), ())

    def key_of(s):
        return s ^ ((s >> 31) & jnp.int32(0x7FFFFFFF))

    def select_bucket(nb, k_cur, total):
        carry = jnp.int32(0)
        na = jnp.int32(0)
        nge = jnp.int32(0)
        for ci in range(nb // _L):
            acc = zero16
            for l in range(_L):
                acc = acc + hist[pl.ds(l * _HS + ci * _L, _L)]
            cum = plsc.cumsum(acc) + carry
            suffix = total - cum + acc
            ge = suffix >= k_cur
            nge = nge + jnp.sum(ge.astype(jnp.int32))
            na = na + jnp.sum(jnp.where(ge, 0, acc))
            carry = carry + jnp.sum(acc)
        return nge - 1, na

    def zero_hist(nb):
        for l in range(_L):
            for ci in range(nb // _L):
                hist[pl.ds(l * _HS + ci * _L, _L)] = zero16

    def level_list(src, dst, shift, nb, lane_cnt, k_cur, sa):
        total = jnp.sum(lane_cnt)
        mx = jnp.max(lane_cnt)
        zero_hist(nb)
        mask_b = jnp.int32(nb - 1)

        @plsc.parallel_loop(0, mx, unroll=4)
        def _(j):
            key = plsc.load_gather(src, [lane_seg + j])
            valid = j < lane_cnt
            bucket = (key >> shift) & mask_b
            plsc.addupdate_scatter(hist, [lane_hist + bucket], ones16,
                                   mask=valid)

        bstar, na = select_bucket(nb, k_cur, total)

        def cbody(j, st):
            cnt, sa = st
            key = plsc.load_gather(src, [lane_seg + j])
            valid = j < lane_cnt
            v = plsc.bitcast(key_of(key), jnp.float32)
            bucket = (key >> shift) & mask_b
            gt = jnp.logical_and(bucket > bstar, valid)
            sa = sa + jnp.where(gt, v, 0.0)
            eq = jnp.logical_and(bucket == bstar, valid)
            plsc.store_scatter(dst, [lane_seg + cnt], key, mask=eq)
            cnt = cnt + eq.astype(jnp.int32)
            return cnt, sa

        cnt, sa = plsc.parallel_loop(0, mx, carry=(zero16, sa),
                                     unroll=4)(cbody)
        return bstar, cnt, k_cur - na, sa

    def select_row(row_ref, c):
        zero_hist(_NB1)

        @plsc.parallel_loop(0, _N // _L, unroll=8)
        def _(j):
            x = row_ref[pl.ds(j * _L, _L)]
            s = plsc.bitcast(x, jnp.int32)
            bucket = (key_of(s) >> 23) + 256
            plsc.addupdate_scatter(hist, [lane_hist + bucket], ones16)

        bstar1, na1 = select_bucket(_NB1, jnp.int32(_K), jnp.int32(_N))

        def c1(j, st):
            cnt, sa = st
            x = row_ref[pl.ds(j * _L, _L)]
            s = plsc.bitcast(x, jnp.int32)
            bucket = (key_of(s) >> 23) + 256
            gt = bucket > bstar1
            sa = sa + jnp.where(gt, x, 0.0)
            eq = bucket == bstar1
            plsc.store_scatter(lista, [lane_seg + cnt], key_of(s), mask=eq)
            cnt = cnt + eq.astype(jnp.int32)
            return cnt, sa

        cnt, sa = plsc.parallel_loop(0, _N // _L, carry=(zero16, zero16f),
                                     unroll=8)(c1)
        kc = jnp.int32(_K) - na1
        k_acc = (bstar1 - 256) << 23
        src, dst = lista, listb
        for shift, nb in _LEVELS:
            bst, cnt, kc, sa = level_list(src, dst, shift, nb, cnt, kc, sa)
            k_acc = k_acc + (bst << shift)
            src, dst = dst, src

        vk = plsc.bitcast(key_of(bcast_i(k_acc)), jnp.float32)
        ties = jnp.where(lane0, bcast_f(kc.astype(jnp.float32)) * vk, 0.0)
        sal = jnp.sum((sa + ties) * (1.0 / _K))
        plsc.store_scatter(salbuf, [bcast_i(c)], bcast_f(sal), mask=lane0)

    pltpu.make_async_copy(fm_hbm.at[b, 0], row0, sem0).start()
    pltpu.sync_copy(pooled_hbm.at[b], pooledbuf)
    pltpu.sync_copy(params_hbm, parbuf)

    def rowpair(i, _):
        c0 = i * 2
        pltpu.make_async_copy(fm_hbm.at[b, c0], row0, sem0).wait()
        cp1 = pltpu.make_async_copy(fm_hbm.at[b, c0 + 1], row1, sem1)
        cp1.start()
        select_row(row0, c0)
        cp1.wait()

        @pl.when(i < _C // 2 - 1)
        def _():
            pltpu.make_async_copy(fm_hbm.at[b, c0 + 2], row0, sem0).start()

        select_row(row1, c0 + 1)
        return 0

    lax.fori_loop(0, _C // 2, rowpair, 0)

    def rsqrt16(v):
        i = plsc.bitcast(v, jnp.int32)
        y = plsc.bitcast(jnp.int32(0x5F3759DF) - (i >> 1), jnp.float32)
        for _ in range(4):
            y = y * (1.5 - 0.5 * v * y * y)
        return y

    nch = _C // _L

    def stats(ref):
        accv = zero16f
        for ci in range(nch):
            accv = accv + ref[pl.ds(ci * _L, _L)]
        mean = bcast_f(jnp.sum(accv) * (1.0 / _C))
        varv = zero16f
        for ci in range(nch):
            d = ref[pl.ds(ci * _L, _L)] - mean
            varv = varv + d * d
        rstd = rsqrt16(bcast_f(jnp.sum(varv) * (1.0 / _C)) + _LN_EPS)
        return mean, rstd

    mean1, rstd1 = stats(salbuf)
    for ci in range(nch):
        sl = pl.ds(ci * _L, _L)
        w1 = parbuf[pl.ds(0 * _C + ci * _L, _L)]
        b1 = parbuf[pl.ds(1 * _C + ci * _L, _L)]
        gs = parbuf[pl.ds(2 * _C + ci * _L, _L)]
        gb = parbuf[pl.ds(3 * _C + ci * _L, _L)]
        rs = parbuf[pl.ds(4 * _C + ci * _L, _L)]
        sal = (salbuf[sl] - mean1) * rstd1 * w1 + b1
        pld = pooledbuf[sl]
        gate = 1.0 / (1.0 + jnp.exp(-(gs * sal + gb)))
        outbuf[sl] = pld + rs * gate * (sal - pld)

    mean2, rstd2 = stats(outbuf)
    for ci in range(nch):
        sl = pl.ds(ci * _L, _L)
        w2 = parbuf[pl.ds(5 * _C + ci * _L, _L)]
        b2 = parbuf[pl.ds(6 * _C + ci * _L, _L)]
        outbuf[sl] = (outbuf[sl] - mean2) * rstd2 * w2 + b2

    pltpu.sync_copy(outbuf, out_hbm.at[b])


_sc_kernel = functools.partial(
    pl.kernel,
    mesh=plsc.VectorSubcoreMesh(core_axis_name="c", subcore_axis_name="s"),
    out_type=jax.ShapeDtypeStruct((_B, _C), jnp.float32),
    compiler_params=pltpu.CompilerParams(needs_layout_passes=False),
    scratch_types=[
        pltpu.VMEM((_N,), jnp.float32),
        pltpu.VMEM((_N,), jnp.float32),
        pltpu.VMEM((_N,), jnp.int32),
        pltpu.VMEM((_N,), jnp.int32),
        pltpu.VMEM((_L * _HS,), jnp.int32),
        pltpu.VMEM((_C,), jnp.float32),
        pltpu.VMEM((_C,), jnp.float32),
        pltpu.VMEM((7 * _C,), jnp.float32),
        pltpu.VMEM((_C,), jnp.float32),
        pltpu.SemaphoreType.DMA,
        pltpu.SemaphoreType.DMA,
    ],
)(_sc_body)


def kernel(pooled, feature_map, ln1_w, ln1_b, gate_scale, gate_bias,
           residual_scale, ln2_w, ln2_b):
    fm = feature_map.astype(jnp.float32).reshape(_B, _C, _N)
    params = jnp.concatenate([
        ln1_w, ln1_b, gate_scale, gate_bias, residual_scale, ln2_w, ln2_b,
    ]).astype(jnp.float32)
    out = _sc_kernel(pooled.astype(jnp.float32), fm, params)
    return out.astype(pooled.dtype)


# speedup vs baseline: 2.8072x; 1.2949x over previous
import functools

import jax
import jax.numpy as jnp
from jax import lax
from jax.experimental import pallas as pl
from jax.experimental.pallas import tpu as pltpu
from jax.experimental.pallas import tpu_sc as plsc

_LN_EPS = 1e-5
_B, _C, _N = 32, 96, 16384
_K = 4096
_L = 16
_SEG = _N // _L
_NB1 = 512
_HS = 512
_LEVELS = [(18, 32), (13, 32), (8, 32), (4, 16), (0, 16)]
_NCORES = 2


def _sc_body(pooled_hbm, fm_hbm, params_hbm, out_hbm,
             row0, row1, lista, listb, hist,
             salbuf, pooledbuf, parbuf, outbuf, sem0, sem1):
    iota = lax.iota(jnp.int32, _L)
    lane_seg = iota * _SEG
    zero16 = jnp.zeros((_L,), jnp.int32)
    zero16f = jnp.zeros((_L,), jnp.float32)
    ones16 = jnp.ones((_L,), jnp.int32)
    lane0 = iota == 0

    b = lax.axis_index("s") * _NCORES + lax.axis_index("c")

    def bcast_i(x):
        return lax.broadcast_in_dim(x, (_L,), ())

    def bcast_f(x):
        return lax.broadcast_in_dim(x, (_L,), ())

    def key_of(s):
        return s ^ ((s >> 31) & jnp.int32(0x7FFFFFFF))

    def select_bucket(nb, k_cur, total):
        carry = jnp.int32(0)
        na_v = zero16
        nge_v = zero16
        for ci in range(nb // _L):
            acc = hist[pl.ds(ci * _L, _L)]
            cum = plsc.cumsum(acc) + carry
            suffix = total - cum + acc
            ge = suffix >= k_cur
            nge_v = nge_v + ge.astype(jnp.int32)
            na_v = na_v + jnp.where(ge, 0, acc)
            carry = carry + jnp.sum(acc)
        return jnp.sum(nge_v) - 1, jnp.sum(na_v)

    def zero_hist(nb):
        for ci in range(nb // _L):
            hist[pl.ds(ci * _L, _L)] = zero16

    def level_list(src, dst, shift, nb, lane_cnt, k_cur, sa):
        total = jnp.sum(lane_cnt)
        mx = jnp.max(lane_cnt)
        zero_hist(nb)
        mask_b = jnp.int32(nb - 1)

        @plsc.parallel_loop(0, mx, unroll=4)
        def _(j):
            key = plsc.load_gather(src, [lane_seg + j])
            valid = j < lane_cnt
            bucket = (key >> shift) & mask_b
            plsc.addupdate_scatter(hist, [bucket], ones16, mask=valid)

        bstar, na = select_bucket(nb, k_cur, total)

        def cbody(j, st):
            cnt, sa = st
            key = plsc.load_gather(src, [lane_seg + j])
            valid = j < lane_cnt
            v = plsc.bitcast(key_of(key), jnp.float32)
            bucket = (key >> shift) & mask_b
            gt = jnp.logical_and(bucket > bstar, valid)
            sa = sa + jnp.where(gt, v, 0.0)
            eq = jnp.logical_and(bucket == bstar, valid)
            plsc.store_scatter(dst, [lane_seg + cnt], key, mask=eq)
            cnt = cnt + eq.astype(jnp.int32)
            return cnt, sa

        cnt, sa = plsc.parallel_loop(0, mx, carry=(zero16, sa),
                                     unroll=4)(cbody)
        return bstar, cnt, k_cur - na, sa

    def select_row(row_ref, c):
        zero_hist(_NB1)

        @plsc.parallel_loop(0, _N // _L, unroll=8)
        def _(j):
            x = row_ref[pl.ds(j * _L, _L)]
            s = plsc.bitcast(x, jnp.int32)
            bucket = (key_of(s) >> 23) + 256
            plsc.addupdate_scatter(hist, [bucket], ones16)

        bstar1, na1 = select_bucket(_NB1, jnp.int32(_K), jnp.int32(_N))

        def c1(j, st):
            cnt, sa = st
            x = row_ref[pl.ds(j * _L, _L)]
            s = plsc.bitcast(x, jnp.int32)
            bucket = (key_of(s) >> 23) + 256
            gt = bucket > bstar1
            sa = sa + jnp.where(gt, x, 0.0)
            eq = bucket == bstar1
            plsc.store_scatter(lista, [lane_seg + cnt], key_of(s), mask=eq)
            cnt = cnt + eq.astype(jnp.int32)
            return cnt, sa

        cnt, sa = plsc.parallel_loop(0, _N // _L, carry=(zero16, zero16f),
                                     unroll=8)(c1)
        kc = jnp.int32(_K) - na1
        k_acc = (bstar1 - 256) << 23
        src, dst = lista, listb
        for shift, nb in _LEVELS:
            bst, cnt, kc, sa = level_list(src, dst, shift, nb, cnt, kc, sa)
            k_acc = k_acc + (bst << shift)
            src, dst = dst, src

        vk = plsc.bitcast(key_of(bcast_i(k_acc)), jnp.float32)
        ties = jnp.where(lane0, bcast_f(kc.astype(jnp.float32)) * vk, 0.0)
        sal = jnp.sum((sa + ties) * (1.0 / _K))
        plsc.store_scatter(salbuf, [bcast_i(c)], bcast_f(sal), mask=lane0)

    pltpu.make_async_copy(fm_hbm.at[b, 0], row0, sem0).start()
    pltpu.sync_copy(pooled_hbm.at[b], pooledbuf)
    pltpu.sync_copy(params_hbm, parbuf)

    def rowpair(i, _):
        c0 = i * 2
        pltpu.make_async_copy(fm_hbm.at[b, c0], row0, sem0).wait()
        cp1 = pltpu.make_async_copy(fm_hbm.at[b, c0 + 1], row1, sem1)
        cp1.start()
        select_row(row0, c0)
        cp1.wait()

        @pl.when(i < _C // 2 - 1)
        def _():
            pltpu.make_async_copy(fm_hbm.at[b, c0 + 2], row0, sem0).start()

        select_row(row1, c0 + 1)
        return 0

    lax.fori_loop(0, _C // 2, rowpair, 0)

    def rsqrt16(v):
        i = plsc.bitcast(v, jnp.int32)
        y = plsc.bitcast(jnp.int32(0x5F3759DF) - (i >> 1), jnp.float32)
        for _ in range(4):
            y = y * (1.5 - 0.5 * v * y * y)
        return y

    nch = _C // _L

    def stats(ref):
        accv = zero16f
        for ci in range(nch):
            accv = accv + ref[pl.ds(ci * _L, _L)]
        mean = bcast_f(jnp.sum(accv) * (1.0 / _C))
        varv = zero16f
        for ci in range(nch):
            d = ref[pl.ds(ci * _L, _L)] - mean
            varv = varv + d * d
        rstd = rsqrt16(bcast_f(jnp.sum(varv) * (1.0 / _C)) + _LN_EPS)
        return mean, rstd

    mean1, rstd1 = stats(salbuf)
    for ci in range(nch):
        sl = pl.ds(ci * _L, _L)
        w1 = parbuf[pl.ds(0 * _C + ci * _L, _L)]
        b1 = parbuf[pl.ds(1 * _C + ci * _L, _L)]
        gs = parbuf[pl.ds(2 * _C + ci * _L, _L)]
        gb = parbuf[pl.ds(3 * _C + ci * _L, _L)]
        rs = parbuf[pl.ds(4 * _C + ci * _L, _L)]
        sal = (salbuf[sl] - mean1) * rstd1 * w1 + b1
        pld = pooledbuf[sl]
        gate = 1.0 / (1.0 + jnp.exp(-(gs * sal + gb)))
        outbuf[sl] = pld + rs * gate * (sal - pld)

    mean2, rstd2 = stats(outbuf)
    for ci in range(nch):
        sl = pl.ds(ci * _L, _L)
        w2 = parbuf[pl.ds(5 * _C + ci * _L, _L)]
        b2 = parbuf[pl.ds(6 * _C + ci * _L, _L)]
        outbuf[sl] = (outbuf[sl] - mean2) * rstd2 * w2 + b2

    pltpu.sync_copy(outbuf, out_hbm.at[b])


_sc_kernel = functools.partial(
    pl.kernel,
    mesh=plsc.VectorSubcoreMesh(core_axis_name="c", subcore_axis_name="s"),
    out_type=jax.ShapeDtypeStruct((_B, _C), jnp.float32),
    compiler_params=pltpu.CompilerParams(needs_layout_passes=False),
    scratch_types=[
        pltpu.VMEM((_N,), jnp.float32),
        pltpu.VMEM((_N,), jnp.float32),
        pltpu.VMEM((_N,), jnp.int32),
        pltpu.VMEM((_N,), jnp.int32),
        pltpu.VMEM((_HS,), jnp.int32),
        pltpu.VMEM((_C,), jnp.float32),
        pltpu.VMEM((_C,), jnp.float32),
        pltpu.VMEM((7 * _C,), jnp.float32),
        pltpu.VMEM((_C,), jnp.float32),
        pltpu.SemaphoreType.DMA,
        pltpu.SemaphoreType.DMA,
    ],
)(_sc_body)


def kernel(pooled, feature_map, ln1_w, ln1_b, gate_scale, gate_bias,
           residual_scale, ln2_w, ln2_b):
    fm = feature_map.astype(jnp.float32).reshape(_B, _C, _N)
    params = jnp.concatenate([
        ln1_w, ln1_b, gate_scale, gate_bias, residual_scale, ln2_w, ln2_b,
    ]).astype(jnp.float32)
    out = _sc_kernel(pooled.astype(jnp.float32), fm, params)
    return out.astype(pooled.dtype)


# speedup vs baseline: 2.8086x; 1.0005x over previous
import functools

import jax
import jax.numpy as jnp
from jax import lax
from jax.experimental import pallas as pl
from jax.experimental.pallas import tpu as pltpu
from jax.experimental.pallas import tpu_sc as plsc

_LN_EPS = 1e-5
_B, _C, _N = 32, 96, 16384
_K = 4096
_L = 16
_SEG = _N // _L
_NB1 = 512
_HS = 512
_LEVELS = [(18, 32), (13, 32), (8, 32), (4, 16), (0, 16)]
_NCORES = 2


def _sc_body(pooled_hbm, fm_hbm, params_hbm, out_hbm,
             row0, row1, lista, listb, hist,
             salbuf, pooledbuf, parbuf, outbuf, sem0, sem1):
    iota = lax.iota(jnp.int32, _L)
    lane_seg = iota * _SEG
    zero16 = jnp.zeros((_L,), jnp.int32)
    zero16f = jnp.zeros((_L,), jnp.float32)
    ones16 = jnp.ones((_L,), jnp.int32)
    lane0 = iota == 0

    b = lax.axis_index("s") * _NCORES + lax.axis_index("c")

    def bcast_i(x):
        return lax.broadcast_in_dim(x, (_L,), ())

    def bcast_f(x):
        return lax.broadcast_in_dim(x, (_L,), ())

    def key_of(s):
        return s ^ ((s >> 31) & jnp.int32(0x7FFFFFFF))

    def select_bucket(nb, k_cur, total):
        carry = jnp.int32(0)
        na_v = zero16
        nge_v = zero16
        for ci in range(nb // _L):
            acc = hist[pl.ds(ci * _L, _L)]
            hist[pl.ds(ci * _L, _L)] = zero16
            local = plsc.cumsum(acc)
            cum = local + carry
            suffix = total - cum + acc
            ge = suffix >= k_cur
            nge_v = nge_v + ge.astype(jnp.int32)
            na_v = na_v + jnp.where(ge, 0, acc)
            carry = carry + local[_L - 1]
        return jnp.sum(nge_v) - 1, jnp.sum(na_v)

    def level_list(src, dst, shift, nb, lane_cnt, k_cur, sa):
        total = jnp.sum(lane_cnt)
        mx = jnp.max(lane_cnt)
        mask_b = jnp.int32(nb - 1)

        @plsc.parallel_loop(0, mx, unroll=4)
        def _(j):
            key = plsc.load_gather(src, [lane_seg + j])
            valid = j < lane_cnt
            bucket = (key >> shift) & mask_b
            plsc.addupdate_scatter(hist, [bucket], ones16, mask=valid)

        bstar, na = select_bucket(nb, k_cur, total)

        def cbody(j, st):
            cnt, sa = st
            key = plsc.load_gather(src, [lane_seg + j])
            valid = j < lane_cnt
            v = plsc.bitcast(key_of(key), jnp.float32)
            bucket = (key >> shift) & mask_b
            gt = jnp.logical_and(bucket > bstar, valid)
            sa = sa + jnp.where(gt, v, 0.0)
            eq = jnp.logical_and(bucket == bstar, valid)
            plsc.store_scatter(dst, [lane_seg + cnt], key, mask=eq)
            cnt = cnt + eq.astype(jnp.int32)
            return cnt, sa

        cnt, sa = plsc.parallel_loop(0, mx, carry=(zero16, sa),
                                     unroll=4)(cbody)
        return bstar, cnt, k_cur - na, sa

    def select_row(row_ref, c):
        @plsc.parallel_loop(0, _N, step=_L, unroll=8)
        def _(j):
            x = row_ref[pl.ds(j, _L)]
            s = plsc.bitcast(x, jnp.int32)
            bucket = (key_of(s) >> 23) + 256
            plsc.addupdate_scatter(hist, [bucket], ones16)

        bstar1, na1 = select_bucket(_NB1, jnp.int32(_K), jnp.int32(_N))

        def c1(j, st):
            cnt, sa = st
            x = row_ref[pl.ds(j, _L)]
            s = plsc.bitcast(x, jnp.int32)
            bucket = (key_of(s) >> 23) + 256
            gt = bucket > bstar1
            sa = sa + jnp.where(gt, x, 0.0)
            eq = bucket == bstar1
            plsc.store_scatter(lista, [lane_seg + cnt], key_of(s), mask=eq)
            cnt = cnt + eq.astype(jnp.int32)
            return cnt, sa

        cnt, sa = plsc.parallel_loop(0, _N, step=_L,
                                     carry=(zero16, zero16f),
                                     unroll=8)(c1)
        kc = jnp.int32(_K) - na1
        k_acc = (bstar1 - 256) << 23
        src, dst = lista, listb
        for shift, nb in _LEVELS:
            bst, cnt, kc, sa = level_list(src, dst, shift, nb, cnt, kc, sa)
            k_acc = k_acc + (bst << shift)
            src, dst = dst, src

        vk = plsc.bitcast(key_of(bcast_i(k_acc)), jnp.float32)
        ties = jnp.where(lane0, bcast_f(kc.astype(jnp.float32)) * vk, 0.0)
        sal = jnp.sum((sa + ties) * (1.0 / _K))
        plsc.store_scatter(salbuf, [bcast_i(c)], bcast_f(sal), mask=lane0)

    pltpu.make_async_copy(fm_hbm.at[b, 0], row0, sem0).start()
    for ci in range(_HS // _L):
        hist[pl.ds(ci * _L, _L)] = zero16
    pltpu.sync_copy(pooled_hbm.at[b], pooledbuf)
    pltpu.sync_copy(params_hbm, parbuf)

    def rowpair(i, _):
        c0 = i * 2
        pltpu.make_async_copy(fm_hbm.at[b, c0], row0, sem0).wait()
        cp1 = pltpu.make_async_copy(fm_hbm.at[b, c0 + 1], row1, sem1)
        cp1.start()
        select_row(row0, c0)
        cp1.wait()

        @pl.when(i < _C // 2 - 1)
        def _():
            pltpu.make_async_copy(fm_hbm.at[b, c0 + 2], row0, sem0).start()

        select_row(row1, c0 + 1)
        return 0

    lax.fori_loop(0, _C // 2, rowpair, 0)

    def rsqrt16(v):
        i = plsc.bitcast(v, jnp.int32)
        y = plsc.bitcast(jnp.int32(0x5F3759DF) - (i >> 1), jnp.float32)
        for _ in range(4):
            y = y * (1.5 - 0.5 * v * y * y)
        return y

    nch = _C // _L

    def stats(ref):
        accv = zero16f
        for ci in range(nch):
            accv = accv + ref[pl.ds(ci * _L, _L)]
        mean = bcast_f(jnp.sum(accv) * (1.0 / _C))
        varv = zero16f
        for ci in range(nch):
            d = ref[pl.ds(ci * _L, _L)] - mean
            varv = varv + d * d
        rstd = rsqrt16(bcast_f(jnp.sum(varv) * (1.0 / _C)) + _LN_EPS)
        return mean, rstd

    mean1, rstd1 = stats(salbuf)
    for ci in range(nch):
        sl = pl.ds(ci * _L, _L)
        w1 = parbuf[pl.ds(0 * _C + ci * _L, _L)]
        b1 = parbuf[pl.ds(1 * _C + ci * _L, _L)]
        gs = parbuf[pl.ds(2 * _C + ci * _L, _L)]
        gb = parbuf[pl.ds(3 * _C + ci * _L, _L)]
        rs = parbuf[pl.ds(4 * _C + ci * _L, _L)]
        sal = (salbuf[sl] - mean1) * rstd1 * w1 + b1
        pld = pooledbuf[sl]
        gate = 1.0 / (1.0 + jnp.exp(-(gs * sal + gb)))
        outbuf[sl] = pld + rs * gate * (sal - pld)

    mean2, rstd2 = stats(outbuf)
    for ci in range(nch):
        sl = pl.ds(ci * _L, _L)
        w2 = parbuf[pl.ds(5 * _C + ci * _L, _L)]
        b2 = parbuf[pl.ds(6 * _C + ci * _L, _L)]
        outbuf[sl] = (outbuf[sl] - mean2) * rstd2 * w2 + b2

    pltpu.sync_copy(outbuf, out_hbm.at[b])


_sc_kernel = functools.partial(
    pl.kernel,
    mesh=plsc.VectorSubcoreMesh(core_axis_name="c", subcore_axis_name="s"),
    out_type=jax.ShapeDtypeStruct((_B, _C), jnp.float32),
    compiler_params=pltpu.CompilerParams(needs_layout_passes=False),
    scratch_types=[
        pltpu.VMEM((_N,), jnp.float32),
        pltpu.VMEM((_N,), jnp.float32),
        pltpu.VMEM((_N,), jnp.int32),
        pltpu.VMEM((_N,), jnp.int32),
        pltpu.VMEM((_HS,), jnp.int32),
        pltpu.VMEM((_C,), jnp.float32),
        pltpu.VMEM((_C,), jnp.float32),
        pltpu.VMEM((7 * _C,), jnp.float32),
        pltpu.VMEM((_C,), jnp.float32),
        pltpu.SemaphoreType.DMA,
        pltpu.SemaphoreType.DMA,
    ],
)(_sc_body)


def kernel(pooled, feature_map, ln1_w, ln1_b, gate_scale, gate_bias,
           residual_scale, ln2_w, ln2_b):
    fm = feature_map.astype(jnp.float32).reshape(_B, _C, _N)
    params = jnp.concatenate([
        ln1_w, ln1_b, gate_scale, gate_bias, residual_scale, ln2_w, ln2_b,
    ]).astype(jnp.float32)
    out = _sc_kernel(pooled.astype(jnp.float32), fm, params)
    return out.astype(pooled.dtype)
